# baseline (speedup 1.0000x reference)
import jax
import jax.numpy as jnp
from jax import lax
from jax.experimental import pallas as pl
from jax.experimental.pallas import tpu as pltpu

VOL_LO = (-40.0, -40.0, -1.0)
VOL_HI = (40.0, 40.0, 5.4)
VOXEL = 0.4
GRID_DIMS = (200, 200, 16)
EPS = 1e-6
G = 32768
F = 8
BIG = 1.0e6
NEG_HALF_LOG2E = -0.7213475204444817

B = 512
NB = G // B
NF = 16
BP = 2048


def _prep_kernel(m_ref, c_ref, o_ref, ftab_ref, itab_ref):
    f32 = jnp.float32
    m = m_ref[...]
    c = c_ref[...]
    op = o_ref[...]
    c00 = c[:, 0:1]; c01 = c[:, 1:2]; c02 = c[:, 2:3]
    c11 = c[:, 4:5]; c12 = c[:, 5:6]; c22 = c[:, 8:9]
    m00 = c11 * c22 - c12 * c12
    m01 = c02 * c12 - c01 * c22
    m02 = c01 * c12 - c02 * c11
    det = c00 * m00 + c01 * m01 + c02 * m02
    rdet = 1.0 / det
    a00 = m00 * rdet
    a01 = m01 * rdet
    a02 = m02 * rdet
    a11 = (c00 * c22 - c02 * c02) * rdet
    a12 = (c01 * c02 - c00 * c12) * rdet
    a22 = (c00 * c11 - c01 * c01) * rdet

    lo = VOL_LO
    hi = VOL_HI
    dims = GRID_DIMS
    ilo = []
    ihi = []
    valid = None
    for t in range(3):
        mt = m[:, t:t + 1]
        ct = (c00, c11, c22)[t]
        sig = jnp.sqrt(ct)
        blo = mt - 3.0 * sig
        bhi = mt + 3.0 * sig
        vt = (bhi > lo[t]) & (blo < hi[t])
        valid = vt if valid is None else (valid & vt)
        bloc = jnp.clip(blo, lo[t], hi[t])
        bhic = jnp.clip(bhi, lo[t], hi[t])
        il = ((bloc - lo[t]) / VOXEL).astype(jnp.int32)
        ih = ((bhic - lo[t]) / VOXEL).astype(jnp.int32)
        ilo.append(jnp.minimum(il, dims[t] - 1))
        ihi.append(jnp.minimum(ih, dims[t] - 1))

    lx = jnp.minimum(ilo[0], dims[0] - 8)
    ly = jnp.minimum(ilo[1], dims[1] - 8)
    lxf = lx.astype(f32)
    lyf = ly.astype(f32)
    x0 = (lxf + 0.5) * VOXEL + lo[0] - m[:, 0:1]
    y0 = (lyf + 0.5) * VOXEL + lo[1] - m[:, 1:2]
    z0 = jnp.zeros_like(x0) + (0.5 * VOXEL + lo[2]) - m[:, 2:3]
    opv = jnp.where(valid, op, 0.0)
    cols = [
        x0, y0, z0,
        a00, a11, a22,
        2.0 * a01, 2.0 * a02, 2.0 * a12,
        opv,
        (ilo[0] - lx).astype(f32), (ihi[0] - lx).astype(f32),
        (ilo[1] - ly).astype(f32), (ihi[1] - ly).astype(f32),
        ilo[2].astype(f32), ihi[2].astype(f32),
    ]
    ftab_ref[...] = jnp.concatenate(cols, axis=1)
    base_pack = lx * 256 + ly
    padi = jnp.zeros_like(base_pack)
    itab_ref[...] = jnp.concatenate([base_pack] + [padi] * 7, axis=1)


def _splat_kernel(itab_ref, ftab_ref, fv_ref, dens_ref, feat_ref,
                  fsm, fsem):
    f32 = jnp.float32
    b = pl.program_id(0)
    slot = lax.rem(b, 2)
    nxt = lax.rem(b + 1, 2)

    @pl.when(b == 0)
    def _():
        dens_ref[...] = jnp.zeros(dens_ref.shape, f32)
        feat_ref[...] = jnp.zeros(feat_ref.shape, f32)
        pltpu.make_async_copy(ftab_ref.at[0], fsm.at[0], fsem.at[0]).start()

    @pl.when(b + 1 < NB)
    def _():
        pltpu.make_async_copy(
            ftab_ref.at[b + 1], fsm.at[nxt], fsem.at[nxt]).start()

    pltpu.make_async_copy(ftab_ref.at[b], fsm.at[slot], fsem.at[slot]).wait()
    base = b * B

    iif = lax.broadcasted_iota(jnp.int32, (8, 8, 128), 0).astype(f32)
    jjf = lax.broadcasted_iota(jnp.int32, (8, 8, 128), 1).astype(f32)
    lli = lax.broadcasted_iota(jnp.int32, (8, 8, 128), 2)
    zzf = (lli >> 3).astype(f32)
    iiv = iif * VOXEL
    jjv = jjf * VOXEL
    zzv = zzf * VOXEL

    def body(g, carry):
        code = itab_ref[base + g]
        lx = code >> 8
        ly = code & 255
        fb = g * NF
        x0 = fsm[slot, 0, fb]
        y0 = fsm[slot, 0, fb + 1]
        z0 = fsm[slot, 0, fb + 2]
        a00 = fsm[slot, 0, fb + 3]
        a11 = fsm[slot, 0, fb + 4]
        a22 = fsm[slot, 0, fb + 5]
        a01 = fsm[slot, 0, fb + 6]
        a02 = fsm[slot, 0, fb + 7]
        a12 = fsm[slot, 0, fb + 8]
        op = fsm[slot, 0, fb + 9]
        bxl = fsm[slot, 0, fb + 10]
        bxh = fsm[slot, 0, fb + 11]
        byl = fsm[slot, 0, fb + 12]
        byh = fsm[slot, 0, fb + 13]
        bzl = fsm[slot, 0, fb + 14]
        bzh = fsm[slot, 0, fb + 15]
        X = jnp.where(iif < bxl, BIG, jnp.where(iif > bxh, BIG, iiv + x0))
        Y = jnp.where(jjf < byl, BIG, jnp.where(jjf > byh, BIG, jjv + y0))
        Z = jnp.where(zzf < bzl, BIG, jnp.where(zzf > bzh, BIG, zzv + z0))
        q = (a00 * X + a01 * Y + a02 * Z) * X + (a11 * Y + a12 * Z) * Y \
            + a22 * Z * Z
        v0 = op * jnp.exp2(q * NEG_HALF_LOG2E)
        fv = v0 * fv_ref[pl.ds(g, 1), :].reshape(1, 1, 128)
        dens_ref[pl.ds(lx, 8), pl.ds(ly, 8), :] += v0
        feat_ref[pl.ds(lx, 8), pl.ds(ly, 8), :] += fv
        return carry

    lax.fori_loop(0, B, body, 0, unroll=4)

    @pl.when(b == NB - 1)
    def _():
        for xi in range(0, 200, 8):
            sl = slice(xi, xi + 8)
            d = dens_ref[sl, :, :]
            feat_ref[sl, :, :] = feat_ref[sl, :, :] / jnp.maximum(d, EPS)


def _run(means3d, opacities, covariances, features, interpret=False):
    cov9 = covariances.reshape(G, 9)
    ftab, itab = pl.pallas_call(
        _prep_kernel,
        grid=(G // BP,),
        in_specs=[
            pl.BlockSpec((BP, 3), lambda i: (i, 0)),
            pl.BlockSpec((BP, 9), lambda i: (i, 0)),
            pl.BlockSpec((BP, 1), lambda i: (i, 0)),
        ],
        out_specs=[
            pl.BlockSpec((BP, 16), lambda i: (i, 0)),
            pl.BlockSpec((BP, 8), lambda i: (i, 0)),
        ],
        out_shape=[
            jax.ShapeDtypeStruct((G, 16), jnp.float32),
            jax.ShapeDtypeStruct((G, 8), jnp.int32),
        ],
        name="voxelizer_prep",
        interpret=interpret,
    )(means3d, cov9, opacities)

    ftab_flat = ftab.reshape(NB, 1, B * NF)
    itab_flat = itab[:, 0]
    featv = jnp.tile(features, (1, 16))

    dens128, feat128 = pl.pallas_call(
        _splat_kernel,
        grid=(NB,),
        in_specs=[
            pl.BlockSpec(memory_space=pltpu.SMEM),
            pl.BlockSpec((NB, 1, B * NF), lambda b: (0, 0, 0)),
            pl.BlockSpec((B, 128), lambda b: (b, 0)),
        ],
        out_specs=[
            pl.BlockSpec((200, 200, 128), lambda b: (0, 0, 0)),
            pl.BlockSpec((200, 200, 128), lambda b: (0, 0, 0)),
        ],
        out_shape=[
            jax.ShapeDtypeStruct((200, 200, 128), jnp.float32),
            jax.ShapeDtypeStruct((200, 200, 128), jnp.float32),
        ],
        scratch_shapes=[
            pltpu.SMEM((2, 1, B * NF), jnp.float32),
            pltpu.SemaphoreType.DMA((2,)),
        ],
        compiler_params=pltpu.CompilerParams(
            dimension_semantics=("arbitrary",),
            vmem_limit_bytes=52 * 1024 * 1024,
        ),
        name="voxelizer_splat",
        interpret=interpret,
    )(itab_flat, ftab_flat, featv)

    grid_feats = feat128.reshape(200, 200, 16, 8)
    grid_density = dens128.reshape(200, 200, 16, 8)[..., 0:1]
    return grid_density, grid_feats


def kernel(means3d, opacities, covariances, features):
    return _run(means3d, opacities, covariances, features)


# speedup vs baseline: 1.2600x; 1.2600x over previous
import jax
import jax.numpy as jnp
from jax import lax
from jax.experimental import pallas as pl
from jax.experimental.pallas import tpu as pltpu

VOL_LO = (-40.0, -40.0, -1.0)
VOL_HI = (40.0, 40.0, 5.4)
VOXEL = 0.4
GRID_DIMS = (200, 200, 16)
EPS = 1e-6
G = 32768
F = 8
BIG = 1.0e6
NEG_HALF_LOG2E = -0.7213475204444817

B = 512
NB = G // B
NF = 16
BP = 2048


def _prep_kernel(m_ref, c_ref, o_ref, ftab_ref, itab_ref):
    f32 = jnp.float32
    m = m_ref[...]
    c = c_ref[...]
    op = o_ref[...]
    c00 = c[:, 0:1]; c01 = c[:, 1:2]; c02 = c[:, 2:3]
    c11 = c[:, 4:5]; c12 = c[:, 5:6]; c22 = c[:, 8:9]
    m00 = c11 * c22 - c12 * c12
    m01 = c02 * c12 - c01 * c22
    m02 = c01 * c12 - c02 * c11
    det = c00 * m00 + c01 * m01 + c02 * m02
    rdet = 1.0 / det
    a00 = m00 * rdet
    a01 = m01 * rdet
    a02 = m02 * rdet
    a11 = (c00 * c22 - c02 * c02) * rdet
    a12 = (c01 * c02 - c00 * c12) * rdet
    a22 = (c00 * c11 - c01 * c01) * rdet

    lo = VOL_LO
    hi = VOL_HI
    dims = GRID_DIMS
    ilo = []
    ihi = []
    valid = None
    for t in range(3):
        mt = m[:, t:t + 1]
        ct = (c00, c11, c22)[t]
        sig = jnp.sqrt(ct)
        blo = mt - 3.0 * sig
        bhi = mt + 3.0 * sig
        vt = (bhi > lo[t]) & (blo < hi[t])
        valid = vt if valid is None else (valid & vt)
        bloc = jnp.clip(blo, lo[t], hi[t])
        bhic = jnp.clip(bhi, lo[t], hi[t])
        il = ((bloc - lo[t]) / VOXEL).astype(jnp.int32)
        ih = ((bhic - lo[t]) / VOXEL).astype(jnp.int32)
        ilo.append(jnp.minimum(il, dims[t] - 1))
        ihi.append(jnp.minimum(ih, dims[t] - 1))

    lx = jnp.minimum(ilo[0], dims[0] - 8)
    ly = jnp.minimum(ilo[1], dims[1] - 8)
    lxf = lx.astype(f32)
    lyf = ly.astype(f32)
    x0 = (lxf + 0.5) * VOXEL + lo[0] - m[:, 0:1]
    y0 = (lyf + 0.5) * VOXEL + lo[1] - m[:, 1:2]
    z0 = jnp.zeros_like(x0) + (0.5 * VOXEL + lo[2]) - m[:, 2:3]
    opv = jnp.where(valid, op, 0.0)
    cols = [
        x0, y0, z0,
        a00, a11, a22,
        2.0 * a01, 2.0 * a02, 2.0 * a12,
        opv,
        (ilo[0] - lx).astype(f32), (ihi[0] - lx).astype(f32),
        (ilo[1] - ly).astype(f32), (ihi[1] - ly).astype(f32),
        ilo[2].astype(f32), ihi[2].astype(f32),
    ]
    ftab_ref[...] = jnp.concatenate(cols, axis=1)
    base_pack = lx * 256 + ly
    padi = jnp.zeros_like(base_pack)
    itab_ref[...] = jnp.concatenate([base_pack] + [padi] * 7, axis=1)


def _splat_kernel(ftab_ref, itab_ref, fv_ref, dens_ref, feat_ref,
                  fsm, ism, fsem, isem):
    f32 = jnp.float32
    b = pl.program_id(0)
    slot = 0

    @pl.when(b == 0)
    def _():
        dens_ref[...] = jnp.zeros(dens_ref.shape, f32)
        feat_ref[...] = jnp.zeros(feat_ref.shape, f32)

    pltpu.make_async_copy(ftab_ref, fsm, fsem.at[slot]).start()
    pltpu.make_async_copy(itab_ref, ism, isem.at[slot]).start()
    pltpu.make_async_copy(ftab_ref, fsm, fsem.at[slot]).wait()
    pltpu.make_async_copy(itab_ref, ism, isem.at[slot]).wait()

    iif = lax.broadcasted_iota(jnp.int32, (8, 8, 128), 0).astype(f32)
    jjf = lax.broadcasted_iota(jnp.int32, (8, 8, 128), 1).astype(f32)
    lli = lax.broadcasted_iota(jnp.int32, (8, 8, 128), 2)
    zzf = (lli >> 3).astype(f32)
    iiv = iif * VOXEL
    jjv = jjf * VOXEL
    zzv = zzf * VOXEL

    def body(g, carry):
        code = ism[0, 0, g * 2]
        lx = code >> 8
        ly = code & 255
        fb = g * NF
        x0 = fsm[0, 0, fb]
        y0 = fsm[0, 0, fb + 1]
        z0 = fsm[0, 0, fb + 2]
        a00 = fsm[0, 0, fb + 3]
        a11 = fsm[0, 0, fb + 4]
        a22 = fsm[0, 0, fb + 5]
        a01 = fsm[0, 0, fb + 6]
        a02 = fsm[0, 0, fb + 7]
        a12 = fsm[0, 0, fb + 8]
        op = fsm[0, 0, fb + 9]
        bxl = fsm[0, 0, fb + 10]
        bxh = fsm[0, 0, fb + 11]
        byl = fsm[0, 0, fb + 12]
        byh = fsm[0, 0, fb + 13]
        bzl = fsm[0, 0, fb + 14]
        bzh = fsm[0, 0, fb + 15]
        X = jnp.where(iif < bxl, BIG, jnp.where(iif > bxh, BIG, iiv + x0))
        Y = jnp.where(jjf < byl, BIG, jnp.where(jjf > byh, BIG, jjv + y0))
        Z = jnp.where(zzf < bzl, BIG, jnp.where(zzf > bzh, BIG, zzv + z0))
        q = (a00 * X + a01 * Y + a02 * Z) * X + (a11 * Y + a12 * Z) * Y \
            + a22 * Z * Z
        v0 = op * jnp.exp2(q * NEG_HALF_LOG2E)
        fv = v0 * fv_ref[pl.ds(g, 1), :].reshape(1, 1, 128)
        dens_ref[pl.ds(lx, 8), pl.ds(ly, 8), :] += v0
        feat_ref[pl.ds(lx, 8), pl.ds(ly, 8), :] += fv
        return carry

    lax.fori_loop(0, B, body, 0, unroll=4)

    @pl.when(b == NB - 1)
    def _():
        for xi in range(0, 200, 8):
            sl = slice(xi, xi + 8)
            d = dens_ref[sl, :, :]
            feat_ref[sl, :, :] = feat_ref[sl, :, :] / jnp.maximum(d, EPS)


def _run(means3d, opacities, covariances, features, interpret=False):
    cov9 = covariances.reshape(G, 9)
    ftab, itab = pl.pallas_call(
        _prep_kernel,
        grid=(G // BP,),
        in_specs=[
            pl.BlockSpec((BP, 3), lambda i: (i, 0)),
            pl.BlockSpec((BP, 9), lambda i: (i, 0)),
            pl.BlockSpec((BP, 1), lambda i: (i, 0)),
        ],
        out_specs=[
            pl.BlockSpec((BP, 16), lambda i: (i, 0)),
            pl.BlockSpec((BP, 8), lambda i: (i, 0)),
        ],
        out_shape=[
            jax.ShapeDtypeStruct((G, 16), jnp.float32),
            jax.ShapeDtypeStruct((G, 8), jnp.int32),
        ],
        name="voxelizer_prep",
        interpret=interpret,
    )(means3d, cov9, opacities)

    ftab_flat = ftab.reshape(NB, 1, B * NF)
    itab_flat = itab[:, :2].reshape(NB, 1, B * 2)
    featv = jnp.tile(features, (1, 16))

    dens128, feat128 = pl.pallas_call(
        _splat_kernel,
        grid=(NB,),
        in_specs=[
            pl.BlockSpec((1, 1, B * NF), lambda b: (b, 0, 0)),
            pl.BlockSpec((1, 1, B * 2), lambda b: (b, 0, 0)),
            pl.BlockSpec((B, 128), lambda b: (b, 0)),
        ],
        out_specs=[
            pl.BlockSpec((200, 200, 128), lambda b: (0, 0, 0)),
            pl.BlockSpec((200, 200, 128), lambda b: (0, 0, 0)),
        ],
        out_shape=[
            jax.ShapeDtypeStruct((200, 200, 128), jnp.float32),
            jax.ShapeDtypeStruct((200, 200, 128), jnp.float32),
        ],
        scratch_shapes=[
            pltpu.SMEM((1, 1, B * NF), jnp.float32),
            pltpu.SMEM((1, 1, B * 2), jnp.int32),
            pltpu.SemaphoreType.DMA((1,)),
            pltpu.SemaphoreType.DMA((1,)),
        ],
        compiler_params=pltpu.CompilerParams(
            dimension_semantics=("arbitrary",),
            vmem_limit_bytes=52 * 1024 * 1024,
        ),
        name="voxelizer_splat",
        interpret=interpret,
    )(ftab_flat, itab_flat, featv)

    grid_feats = feat128.reshape(200, 200, 16, 8)
    grid_density = dens128.reshape(200, 200, 16, 8)[..., 0:1]
    return grid_density, grid_feats


def kernel(means3d, opacities, covariances, features):
    return _run(means3d, opacities, covariances, features)


# speedup vs baseline: 1.5055x; 1.1948x over previous
import jax
import jax.numpy as jnp
from jax import lax
from jax.experimental import pallas as pl
from jax.experimental.pallas import tpu as pltpu

VOL_LO = (-40.0, -40.0, -1.0)
VOL_HI = (40.0, 40.0, 5.4)
VOXEL = 0.4
GRID_DIMS = (200, 200, 16)
EPS = 1e-6
G = 32768
F = 8
BIG = 1.0e6
NEG_HALF_LOG2E = -0.7213475204444817

B = 512
NB = G // B
NF = 16
BP = 2048


def _prep_kernel(m_ref, c_ref, o_ref, ftab_ref, itab_ref):
    f32 = jnp.float32
    c00 = c_ref[0:1, :]; c01 = c_ref[1:2, :]; c02 = c_ref[2:3, :]
    c11 = c_ref[4:5, :]; c12 = c_ref[5:6, :]; c22 = c_ref[8:9, :]
    op = o_ref[0:1, :]
    m00 = c11 * c22 - c12 * c12
    m01 = c02 * c12 - c01 * c22
    m02 = c01 * c12 - c02 * c11
    det = c00 * m00 + c01 * m01 + c02 * m02
    rdet = 1.0 / det
    a00 = m00 * rdet
    a01 = m01 * rdet
    a02 = m02 * rdet
    a11 = (c00 * c22 - c02 * c02) * rdet
    a12 = (c01 * c02 - c00 * c12) * rdet
    a22 = (c00 * c11 - c01 * c01) * rdet

    lo = VOL_LO
    hi = VOL_HI
    dims = GRID_DIMS
    ilo = []
    ihi = []
    valid = None
    for t in range(3):
        mt = m_ref[t:t + 1, :]
        ct = (c00, c11, c22)[t]
        sig = jnp.sqrt(ct)
        blo = mt - 3.0 * sig
        bhi = mt + 3.0 * sig
        vt = (bhi > lo[t]) & (blo < hi[t])
        valid = vt if valid is None else (valid & vt)
        bloc = jnp.clip(blo, lo[t], hi[t])
        bhic = jnp.clip(bhi, lo[t], hi[t])
        il = ((bloc - lo[t]) / VOXEL).astype(jnp.int32)
        ih = ((bhic - lo[t]) / VOXEL).astype(jnp.int32)
        ilo.append(jnp.minimum(il, dims[t] - 1))
        ihi.append(jnp.minimum(ih, dims[t] - 1))

    lx = jnp.minimum(ilo[0], dims[0] - 8)
    ly = jnp.minimum(ilo[1], dims[1] - 8)
    lxf = lx.astype(f32)
    lyf = ly.astype(f32)
    rows = [
        (lxf + 0.5) * VOXEL + lo[0] - m_ref[0:1, :],
        (lyf + 0.5) * VOXEL + lo[1] - m_ref[1:2, :],
        (0.5 * VOXEL + lo[2]) - m_ref[2:3, :],
        a00, a11, a22,
        2.0 * a01, 2.0 * a02, 2.0 * a12,
        jnp.where(valid, op, 0.0),
        (ilo[0] - lx).astype(f32), (ihi[0] - lx).astype(f32),
        (ilo[1] - ly).astype(f32), (ihi[1] - ly).astype(f32),
        ilo[2].astype(f32), ihi[2].astype(f32),
    ]
    for k, r in enumerate(rows):
        ftab_ref[k:k + 1, :] = r
    itab_ref[0:1, :] = lx * 256 + ly


def _splat_kernel(ftab_ref, itab_ref, fv_ref, dens_ref, feat_ref,
                  fsm, ism, fsem, isem):
    f32 = jnp.float32
    b = pl.program_id(0)
    slot = 0

    @pl.when(b == 0)
    def _():
        dens_ref[...] = jnp.zeros(dens_ref.shape, f32)
        feat_ref[...] = jnp.zeros(feat_ref.shape, f32)

    pltpu.make_async_copy(ftab_ref, fsm, fsem.at[slot]).start()
    pltpu.make_async_copy(itab_ref, ism, isem.at[slot]).start()
    pltpu.make_async_copy(ftab_ref, fsm, fsem.at[slot]).wait()
    pltpu.make_async_copy(itab_ref, ism, isem.at[slot]).wait()

    iif = lax.broadcasted_iota(jnp.int32, (8, 8, 128), 0).astype(f32)
    jjf = lax.broadcasted_iota(jnp.int32, (8, 8, 128), 1).astype(f32)
    lli = lax.broadcasted_iota(jnp.int32, (8, 8, 128), 2)
    zzf = (lli >> 3).astype(f32)
    iiv = iif * VOXEL
    jjv = jjf * VOXEL
    zzv = zzf * VOXEL

    def body(g, carry):
        code = ism[0, 0, g]
        lx = code >> 8
        ly = code & 255
        fb = g * NF
        x0 = fsm[0, 0, fb]
        y0 = fsm[0, 0, fb + 1]
        z0 = fsm[0, 0, fb + 2]
        a00 = fsm[0, 0, fb + 3]
        a11 = fsm[0, 0, fb + 4]
        a22 = fsm[0, 0, fb + 5]
        a01 = fsm[0, 0, fb + 6]
        a02 = fsm[0, 0, fb + 7]
        a12 = fsm[0, 0, fb + 8]
        op = fsm[0, 0, fb + 9]
        bxl = fsm[0, 0, fb + 10]
        bxh = fsm[0, 0, fb + 11]
        byl = fsm[0, 0, fb + 12]
        byh = fsm[0, 0, fb + 13]
        bzl = fsm[0, 0, fb + 14]
        bzh = fsm[0, 0, fb + 15]
        X = jnp.where(iif < bxl, BIG, jnp.where(iif > bxh, BIG, iiv + x0))
        Y = jnp.where(jjf < byl, BIG, jnp.where(jjf > byh, BIG, jjv + y0))
        Z = jnp.where(zzf < bzl, BIG, jnp.where(zzf > bzh, BIG, zzv + z0))
        q = (a00 * X + a01 * Y + a02 * Z) * X + (a11 * Y + a12 * Z) * Y \
            + a22 * Z * Z
        v0 = op * jnp.exp2(q * NEG_HALF_LOG2E)
        fv = v0 * fv_ref[pl.ds(g, 1), :].reshape(1, 1, 128)
        dens_ref[pl.ds(lx, 8), pl.ds(ly, 8), :] += v0
        feat_ref[pl.ds(lx, 8), pl.ds(ly, 8), :] += fv
        return carry

    lax.fori_loop(0, B, body, 0, unroll=4)

    @pl.when(b == NB - 1)
    def _():
        for xi in range(0, 200, 8):
            sl = slice(xi, xi + 8)
            d = dens_ref[sl, :, :]
            feat_ref[sl, :, :] = feat_ref[sl, :, :] / jnp.maximum(d, EPS)


def _run(means3d, opacities, covariances, features, interpret=False):
    cov9t = covariances.reshape(G, 9).T
    m3t = means3d.T
    o1t = opacities.T
    ftab, itab = pl.pallas_call(
        _prep_kernel,
        grid=(G // BP,),
        in_specs=[
            pl.BlockSpec((3, BP), lambda i: (0, i)),
            pl.BlockSpec((9, BP), lambda i: (0, i)),
            pl.BlockSpec((1, BP), lambda i: (0, i)),
        ],
        out_specs=[
            pl.BlockSpec((16, BP), lambda i: (0, i)),
            pl.BlockSpec((1, BP), lambda i: (0, i)),
        ],
        out_shape=[
            jax.ShapeDtypeStruct((16, G), jnp.float32),
            jax.ShapeDtypeStruct((1, G), jnp.int32),
        ],
        name="voxelizer_prep",
        interpret=interpret,
    )(m3t, cov9t, o1t)

    ftab_flat = ftab.T.reshape(NB, 1, B * NF)
    itab_flat = itab[0].reshape(NB, 1, B)
    featv = jnp.tile(features, (1, 16))

    dens128, feat128 = pl.pallas_call(
        _splat_kernel,
        grid=(NB,),
        in_specs=[
            pl.BlockSpec((1, 1, B * NF), lambda b: (b, 0, 0)),
            pl.BlockSpec((1, 1, B), lambda b: (b, 0, 0)),
            pl.BlockSpec((B, 128), lambda b: (b, 0)),
        ],
        out_specs=[
            pl.BlockSpec((200, 200, 128), lambda b: (0, 0, 0)),
            pl.BlockSpec((200, 200, 128), lambda b: (0, 0, 0)),
        ],
        out_shape=[
            jax.ShapeDtypeStruct((200, 200, 128), jnp.float32),
            jax.ShapeDtypeStruct((200, 200, 128), jnp.float32),
        ],
        scratch_shapes=[
            pltpu.SMEM((1, 1, B * NF), jnp.float32),
            pltpu.SMEM((1, 1, B), jnp.int32),
            pltpu.SemaphoreType.DMA((1,)),
            pltpu.SemaphoreType.DMA((1,)),
        ],
        compiler_params=pltpu.CompilerParams(
            dimension_semantics=("arbitrary",),
            vmem_limit_bytes=52 * 1024 * 1024,
        ),
        name="voxelizer_splat",
        interpret=interpret,
    )(ftab_flat, itab_flat, featv)

    grid_feats = feat128.reshape(200, 200, 16, 8)
    grid_density = dens128.reshape(200, 200, 16, 8)[..., 0:1]
    return grid_density, grid_feats


def kernel(means3d, opacities, covariances, features):
    return _run(means3d, opacities, covariances, features)
